# Initial kernel scaffold; baseline (speedup 1.0000x reference)
import jax
import jax.numpy as jnp
from jax.experimental import pallas as pl


def kernel(v, faces):
    raise NotImplementedError("write your pallas kernel here")


# speedup vs baseline: 13.1702x; 13.1702x over previous
import functools

import jax
import jax.numpy as jnp
from jax import lax
from jax.experimental import pallas as pl
from jax.experimental.pallas import tpu as pltpu
from jax.experimental.pallas import tpu_sc as plsc

B = 32
N_VERTS = 100000
N_FACES = 200000
NC, NS, L = 2, 16, 16
NW = NC * NS
CHUNK = 128
NCHUNK = 50
FACES_PAD = NW * NCHUNK * CHUNK
D = B * 3
NCOL = D // L


def _sc_body(vt_hbm, idx_hbm, out_hbm, idx_v, buf_v, out_stage, sem0, sem1):
    wid = lax.axis_index("s") * NC + lax.axis_index("c")
    sems = (sem0, sem1)

    pltpu.sync_copy(idx_hbm.at[wid], idx_v)

    def fire(k, slot):
        for g in range(3):
            pltpu.async_copy(vt_hbm.at[idx_v.at[g, k]], buf_v.at[slot, g],
                             sems[slot])

    def drain(k, slot):
        for g in range(3):
            pltpu.make_async_copy(vt_hbm.at[idx_v.at[g, k]],
                                  buf_v.at[slot, g], sems[slot]).wait()

    def compute(slot, accs):
        def face_body(j, accs):
            accs = list(accs)
            for c in range(NCOL):
                a = buf_v[slot, 0, j, pl.ds(L * c, L)]
                b = buf_v[slot, 1, j, pl.ds(L * c, L)]
                d = buf_v[slot, 2, j, pl.ds(L * c, L)]
                hi = jnp.maximum(a, jnp.maximum(b, d))
                lo = jnp.minimum(a, jnp.minimum(b, d))
                accs[c] = accs[c] + (hi - lo)
            return tuple(accs)
        return lax.fori_loop(0, CHUNK, face_body, accs)

    fire(0, 0)
    zero = jnp.zeros((L,), jnp.float32)
    accs = (zero,) * NCOL

    def pair_body(i, accs):
        for b in range(2):
            k = 2 * i + b
            slot = b
            @pl.when(k + 1 < NCHUNK)
            def _():
                fire(k + 1, 1 - slot)
            drain(k, slot)
            accs = compute(slot, accs)
        return accs

    accs = lax.fori_loop(0, NCHUNK // 2, pair_body, accs)

    total = accs[0]
    for c in range(1, NCOL):
        total = total + accs[c]
    out_stage[...] = total
    pltpu.sync_copy(out_stage, out_hbm.at[wid])


@functools.partial(
    pl.kernel,
    out_type=jax.ShapeDtypeStruct((NW, L), jnp.float32),
    mesh=plsc.VectorSubcoreMesh(core_axis_name="c", subcore_axis_name="s"),
    compiler_params=pltpu.CompilerParams(use_tc_tiling_on_sc=False),
    scratch_types=[
        pltpu.VMEM((3, NCHUNK, CHUNK), jnp.int32),
        pltpu.VMEM((2, 3, CHUNK, D), jnp.float32),
        pltpu.VMEM((L,), jnp.float32),
        pltpu.SemaphoreType.DMA,
        pltpu.SemaphoreType.DMA,
    ],
)
def _edge_loss_sc(vt_hbm, idx_hbm, out_hbm, idx_v, buf_v, out_stage,
                  sem0, sem1):
    _sc_body(vt_hbm, idx_hbm, out_hbm, idx_v, buf_v, out_stage, sem0, sem1)


def kernel(v, faces):
    vt = jnp.transpose(v, (1, 0, 2)).reshape(N_VERTS, D)
    fi = faces.astype(jnp.int32).T
    fi = jnp.pad(fi, ((0, 0), (0, FACES_PAD - N_FACES)))
    fi = fi.reshape(3, NW, NCHUNK, CHUNK).transpose(1, 0, 2, 3)
    partials = _edge_loss_sc(vt, fi)
    return 2.0 * jnp.sum(partials) / jnp.float32(B * N_FACES * 3)


# speedup vs baseline: 15.1489x; 1.1502x over previous
import functools

import jax
import jax.numpy as jnp
from jax import lax
from jax.experimental import pallas as pl
from jax.experimental.pallas import tpu as pltpu
from jax.experimental.pallas import tpu_sc as plsc

B = 32
N_VERTS = 100000
N_FACES = 200000
NC, NS, L = 2, 16, 16
NW = NC * NS
CHUNK = 128
NCHUNK = 50
FACES_PAD = NW * NCHUNK * CHUNK
D = B * 3
DP = 128
NCOL = D // L


def _sc_body(vt_hbm, idx_hbm, out_hbm, idx_v, buf_v, out_stage, sem0, sem1):
    wid = lax.axis_index("s") * NC + lax.axis_index("c")
    sems = (sem0, sem1)

    pltpu.sync_copy(idx_hbm.at[wid], idx_v)

    def fire(k, slot):
        for g in range(3):
            pltpu.async_copy(vt_hbm.at[idx_v.at[g, k]], buf_v.at[slot, g],
                             sems[slot])

    def drain(k, slot):
        for g in range(3):
            pltpu.make_async_copy(vt_hbm.at[idx_v.at[g, k]],
                                  buf_v.at[slot, g], sems[slot]).wait()

    def compute(slot, accs):
        def face_body(j, accs):
            accs = list(accs)
            for c in range(NCOL):
                a = buf_v[slot, 0, j, pl.ds(L * c, L)]
                b = buf_v[slot, 1, j, pl.ds(L * c, L)]
                d = buf_v[slot, 2, j, pl.ds(L * c, L)]
                hi = jnp.maximum(a, jnp.maximum(b, d))
                lo = jnp.minimum(a, jnp.minimum(b, d))
                accs[c] = accs[c] + (hi - lo)
            return tuple(accs)
        return lax.fori_loop(0, CHUNK, face_body, accs)

    fire(0, 0)
    zero = jnp.zeros((L,), jnp.float32)
    accs = (zero,) * NCOL

    def pair_body(i, accs):
        for b in range(2):
            k = 2 * i + b
            slot = b
            @pl.when(k + 1 < NCHUNK)
            def _():
                fire(k + 1, 1 - slot)
            drain(k, slot)
            accs = compute(slot, accs)
        return accs

    accs = lax.fori_loop(0, NCHUNK // 2, pair_body, accs)

    total = accs[0]
    for c in range(1, NCOL):
        total = total + accs[c]
    out_stage[...] = total
    pltpu.sync_copy(out_stage, out_hbm.at[wid])


@functools.partial(
    pl.kernel,
    out_type=jax.ShapeDtypeStruct((NW, L), jnp.float32),
    mesh=plsc.VectorSubcoreMesh(core_axis_name="c", subcore_axis_name="s"),
    compiler_params=pltpu.CompilerParams(use_tc_tiling_on_sc=True),
    scratch_types=[
        pltpu.VMEM((3, NCHUNK, CHUNK), jnp.int32),
        pltpu.VMEM((2, 3, CHUNK, DP), jnp.float32),
        pltpu.VMEM((L,), jnp.float32),
        pltpu.SemaphoreType.DMA,
        pltpu.SemaphoreType.DMA,
    ],
)
def _edge_loss_sc(vt_hbm, idx_hbm, out_hbm, idx_v, buf_v, out_stage,
                  sem0, sem1):
    _sc_body(vt_hbm, idx_hbm, out_hbm, idx_v, buf_v, out_stage, sem0, sem1)


def kernel(v, faces):
    vt = jnp.transpose(v, (1, 0, 2)).reshape(N_VERTS, D)
    vt = jnp.pad(vt, ((0, 0), (0, DP - D)))
    fi = faces.astype(jnp.int32).T
    fi = jnp.pad(fi, ((0, 0), (0, FACES_PAD - N_FACES)))
    fi = fi.reshape(3, NW, NCHUNK, CHUNK).transpose(1, 0, 2, 3)
    partials = _edge_loss_sc(vt, fi)
    return 2.0 * jnp.sum(partials) / jnp.float32(B * N_FACES * 3)


# speedup vs baseline: 15.4116x; 1.0173x over previous
import functools

import jax
import jax.numpy as jnp
import numpy as np
from jax import lax
from jax.experimental import pallas as pl
from jax.experimental.pallas import tpu as pltpu
from jax.experimental.pallas import tpu_sc as plsc

B = 32
N_VERTS = 100000
N_FACES = 200000
NC, NS, L = 2, 16, 16
NW = NC * NS
CHUNK = 120
NCHUNK_TOT = 1696
FACES_PAD = NCHUNK_TOT * CHUNK
W0, W1 = 88, 18
WMAX = W0
D = B * 3
DP = 128
NCOL = D // L


def _worker_chunk_ids():
    ids = np.zeros((NW, WMAX), dtype=np.int32)
    for wid in range(NW):
        c, s = wid % NC, wid // NC
        if c == 0:
            start, count = s * W0, W0
        else:
            start, count = NS * W0 + s * W1, W1
        row = list(range(start, start + count))
        row += [row[-1]] * (WMAX - count)
        ids[wid] = row
    return ids


def _sc_body(vt_hbm, idx_hbm, out_hbm, idx_v, buf_v, out_stage, sem0, sem1):
    cid = lax.axis_index("c")
    wid = lax.axis_index("s") * NC + cid
    count = jnp.where(cid == 0, W0, W1)
    sems = (sem0, sem1)

    pltpu.sync_copy(idx_hbm.at[wid], idx_v)

    def fire(k, slot):
        for g in range(3):
            pltpu.async_copy(vt_hbm.at[idx_v.at[g, k]], buf_v.at[slot, g],
                             sems[slot])

    def drain(k, slot):
        for g in range(3):
            pltpu.make_async_copy(vt_hbm.at[idx_v.at[g, k]],
                                  buf_v.at[slot, g], sems[slot]).wait()

    def compute(slot, accs):
        def face_body(j, accs):
            accs = list(accs)
            for c in range(NCOL):
                a = buf_v[slot, 0, j, pl.ds(L * c, L)]
                b = buf_v[slot, 1, j, pl.ds(L * c, L)]
                d = buf_v[slot, 2, j, pl.ds(L * c, L)]
                hi = jnp.maximum(a, jnp.maximum(b, d))
                lo = jnp.minimum(a, jnp.minimum(b, d))
                accs[c] = accs[c] + (hi - lo)
            return tuple(accs)
        return lax.fori_loop(0, CHUNK, face_body, accs)

    fire(0, 0)
    zero = jnp.zeros((L,), jnp.float32)
    accs = (zero,) * NCOL

    def chunk_step(k, slot, accs):
        @pl.when(k + 1 < count)
        def _():
            fire(k + 1, 1 - slot)
        drain(k, slot)
        return compute(slot, accs)

    def pair_body(i, accs):
        accs = chunk_step(2 * i, 0, accs)
        accs = chunk_step(2 * i + 1, 1, accs)
        return accs

    accs = lax.fori_loop(0, count // 2, pair_body, accs)

    total = accs[0]
    for c in range(1, NCOL):
        total = total + accs[c]
    out_stage[...] = total
    pltpu.sync_copy(out_stage, out_hbm.at[wid])


@functools.partial(
    pl.kernel,
    out_type=jax.ShapeDtypeStruct((NW, L), jnp.float32),
    mesh=plsc.VectorSubcoreMesh(core_axis_name="c", subcore_axis_name="s"),
    compiler_params=pltpu.CompilerParams(use_tc_tiling_on_sc=True),
    scratch_types=[
        pltpu.VMEM((3, WMAX, CHUNK), jnp.int32),
        pltpu.VMEM((2, 3, CHUNK, DP), jnp.float32),
        pltpu.VMEM((L,), jnp.float32),
        pltpu.SemaphoreType.DMA,
        pltpu.SemaphoreType.DMA,
    ],
)
def _edge_loss_sc(vt_hbm, idx_hbm, out_hbm, idx_v, buf_v, out_stage,
                  sem0, sem1):
    _sc_body(vt_hbm, idx_hbm, out_hbm, idx_v, buf_v, out_stage, sem0, sem1)


def kernel(v, faces):
    vt = jnp.transpose(v, (1, 0, 2)).reshape(N_VERTS, D)
    vt = jnp.pad(vt, ((0, 0), (0, DP - D)))
    fi = faces.astype(jnp.int32).T
    fi = jnp.pad(fi, ((0, 0), (0, FACES_PAD - N_FACES)))
    fi = fi.reshape(3, NCHUNK_TOT, CHUNK)
    fi = fi[:, _worker_chunk_ids(), :]
    fi = fi.transpose(1, 0, 2, 3)
    partials = _edge_loss_sc(vt, fi)
    return 2.0 * jnp.sum(partials) / jnp.float32(B * N_FACES * 3)


# speedup vs baseline: 24.7606x; 1.6066x over previous
import functools

import jax
import jax.numpy as jnp
from jax import lax
from jax.experimental import pallas as pl
from jax.experimental.pallas import tpu as pltpu
from jax.experimental.pallas import tpu_sc as plsc

B = 32
N_VERTS = 100000
N_FACES = 200000
NC, NS, L = 2, 16, 16
CHUNK = 128
NCHUNK = 98
FACES_PAD = NS * NCHUNK * CHUNK
D = B * 3
NCOL = D // L


def _sc_body(vt_hbm, idx_hbm, out_hbm, idx_v, buf_v, out_stage, sem0, sem1):
    cid = lax.axis_index("c")
    sid = lax.axis_index("s")
    wid = sid * NC + cid
    sems = (sem0, sem1)

    @pl.when(cid == 0)
    def _work():
        pltpu.sync_copy(idx_hbm.at[sid], idx_v)

        def fire(k, slot):
            for g in range(3):
                pltpu.async_copy(vt_hbm.at[idx_v.at[g, k]],
                                 buf_v.at[slot, g], sems[slot])

        def drain(k, slot):
            for g in range(3):
                pltpu.make_async_copy(vt_hbm.at[idx_v.at[g, k]],
                                      buf_v.at[slot, g], sems[slot]).wait()

        def compute(slot, accs):
            def face_body(j, accs):
                accs = list(accs)
                for c in range(NCOL):
                    a = buf_v[slot, 0, j, pl.ds(L * c, L)]
                    b = buf_v[slot, 1, j, pl.ds(L * c, L)]
                    d = buf_v[slot, 2, j, pl.ds(L * c, L)]
                    hi = jnp.maximum(a, jnp.maximum(b, d))
                    lo = jnp.minimum(a, jnp.minimum(b, d))
                    accs[c] = accs[c] + (hi - lo)
                return tuple(accs)
            return lax.fori_loop(0, CHUNK, face_body, accs)

        fire(0, 0)
        zero = jnp.zeros((L,), jnp.float32)
        accs = (zero,) * NCOL

        def chunk_step(k, slot, accs):
            @pl.when(k + 1 < NCHUNK)
            def _():
                fire(k + 1, 1 - slot)
            drain(k, slot)
            return compute(slot, accs)

        def pair_body(i, accs):
            accs = chunk_step(2 * i, 0, accs)
            accs = chunk_step(2 * i + 1, 1, accs)
            return accs

        accs = lax.fori_loop(0, NCHUNK // 2, pair_body, accs)

        total = accs[0]
        for c in range(1, NCOL):
            total = total + accs[c]
        out_stage[...] = total
        pltpu.sync_copy(out_stage, out_hbm.at[wid])

    @pl.when(cid != 0)
    def _idle():
        out_stage[...] = jnp.zeros((L,), jnp.float32)
        pltpu.sync_copy(out_stage, out_hbm.at[wid])


@functools.partial(
    pl.kernel,
    out_type=jax.ShapeDtypeStruct((NC * NS, L), jnp.float32),
    mesh=plsc.VectorSubcoreMesh(core_axis_name="c", subcore_axis_name="s"),
    compiler_params=pltpu.CompilerParams(use_tc_tiling_on_sc=False),
    scratch_types=[
        pltpu.VMEM((3, NCHUNK, CHUNK), jnp.int32),
        pltpu.VMEM((2, 3, CHUNK, D), jnp.float32),
        pltpu.VMEM((L,), jnp.float32),
        pltpu.SemaphoreType.DMA,
        pltpu.SemaphoreType.DMA,
    ],
)
def _edge_loss_sc(vt_hbm, idx_hbm, out_hbm, idx_v, buf_v, out_stage,
                  sem0, sem1):
    _sc_body(vt_hbm, idx_hbm, out_hbm, idx_v, buf_v, out_stage, sem0, sem1)


def kernel(v, faces):
    vt = jnp.transpose(v, (1, 0, 2)).reshape(N_VERTS, D)
    fi = faces.astype(jnp.int32).T
    fi = jnp.pad(fi, ((0, 0), (0, FACES_PAD - N_FACES)))
    fi = fi.reshape(3, NS, NCHUNK, CHUNK).transpose(1, 0, 2, 3)
    partials = _edge_loss_sc(vt, fi)
    return 2.0 * jnp.sum(partials) / jnp.float32(B * N_FACES * 3)
